# baseline (speedup 1.0000x reference)
import functools

import jax
import jax.numpy as jnp
from jax import lax
from jax.experimental import pallas as pl
from jax.experimental.pallas import tpu as pltpu
from jax.experimental.pallas import tpu_sc as plsc

K = 1024
D = 64
B = 16
HW = 1024
T = 1024
C = HW // T
NTOK = B * HW
NSTEP = B * C
COMMIT = 0.25

NC = 2
NS = 16
NW = NC * NS
TPT = NTOK // NW
M = NTOK * K
EPT = M // NW
ZCHUNK = 16384
NZDMA = EPT // ZCHUNK


def _vq_tc_body(x_ref, emb_ref, idx_ref, q_ref, loss_ref, perp_ref,
                counts_ref, sse_ref):
    i = pl.program_id(0)

    @pl.when(i == 0)
    def _init():
        counts_ref[...] = jnp.zeros_like(counts_ref)
        sse_ref[0] = 0.0

    x = x_ref[0]
    x2 = jnp.sum(x ** 2, axis=0)
    xd = x + x
    emb = emb_ref[...]
    e2 = jnp.sum(emb ** 2, axis=1)
    xe2 = lax.dot_general(xd, emb, (((0,), (1,)), ((), ())),
                          preferred_element_type=jnp.float32)
    scores = (x2[:, None] + e2[None, :]) - xe2
    minval = jnp.min(scores, axis=1)
    iota_k = lax.broadcasted_iota(jnp.int32, (T, K), 1)
    idx = jnp.min(jnp.where(scores == minval[:, None], iota_k, K), axis=1)
    idx_ref[0, 0] = idx
    enc = (iota_k == idx[:, None]).astype(jnp.float32)
    counts_ref[...] += jnp.sum(enc, axis=0)
    qT = lax.dot_general(emb, enc, (((0,), (1,)), ((), ())),
                         preferred_element_type=jnp.float32)
    q_ref[0] = qT
    diff = qT - x
    sse_ref[0] += jnp.sum(diff * diff)

    @pl.when(i == NSTEP - 1)
    def _fini():
        loss_ref[0, 0] = (1.0 + COMMIT) * sse_ref[0] / (NTOK * D)
        avg = counts_ref[...] * (1.0 / NTOK)
        perp_ref[0, 0] = jnp.exp(-jnp.sum(avg * jnp.log(avg + 1e-10)))


_SC_MESH = plsc.VectorSubcoreMesh(core_axis_name="c", subcore_axis_name="s")


@functools.partial(
    pl.kernel,
    out_type=jax.ShapeDtypeStruct((M,), jnp.float32),
    mesh=_SC_MESH,
    scratch_types=[pltpu.VMEM((ZCHUNK,), jnp.float32)],
)
def _sc_fill(out_hbm, zbuf):
    wid = lax.axis_index("s") * NC + lax.axis_index("c")
    base = wid * EPT

    def _zero(j, carry):
        zbuf[pl.ds(j * 16, 16)] = jnp.zeros((16,), jnp.float32)
        return carry

    lax.fori_loop(0, ZCHUNK // 16, _zero, 0)

    def _dma(t, carry):
        pltpu.sync_copy(zbuf, out_hbm.at[pl.ds(base + t * ZCHUNK, ZCHUNK)])
        return carry

    lax.fori_loop(0, NZDMA, _dma, 0)


@functools.partial(
    pl.kernel,
    out_type=(),
    mesh=_SC_MESH,
    scratch_types=[
        pltpu.VMEM((TPT,), jnp.int32),
        pltpu.VMEM((4, 128), jnp.int32),
        pltpu.VMEM((128,), jnp.float32),
        pltpu.SemaphoreType.DMA,
    ],
)
def _sc_scatter(enc_ref, idx_hbm, idxbuf, offbuf, ones, sem):
    wid = lax.axis_index("s") * NC + lax.axis_index("c")
    base = wid * TPT
    pltpu.sync_copy(idx_hbm.at[pl.ds(base, TPT)], idxbuf)
    for j in range(8):
        ones[pl.ds(j * 16, 16)] = jnp.ones((16,), jnp.float32)
    lane_k = lax.iota(jnp.int32, 16) * K
    for j in range(TPT // 16):
        iv = idxbuf[pl.ds(j * 16, 16)]
        off = iv + lane_k + ((base + j * 16) * K)
        offbuf[j // 8, pl.ds((j % 8) * 16, 16)] = off
    for c in range(4):
        pltpu.async_copy(ones, enc_ref.at[offbuf.at[c]], sem).wait()


def kernel(inputs, embedding):
    xr = inputs.reshape(B, D, HW)
    idx3, q, loss, perp = pl.pallas_call(
        _vq_tc_body,
        grid=(NSTEP,),
        in_specs=[
            pl.BlockSpec((1, D, T), lambda i: (i // C, 0, i % C)),
            pl.BlockSpec((K, D), lambda i: (0, 0)),
        ],
        out_specs=[
            pl.BlockSpec((1, 1, T), lambda i: (i, 0, 0)),
            pl.BlockSpec((1, D, T), lambda i: (i // C, 0, i % C)),
            pl.BlockSpec((1, 1), lambda i: (0, 0), memory_space=pltpu.SMEM),
            pl.BlockSpec((1, 1), lambda i: (0, 0), memory_space=pltpu.SMEM),
        ],
        out_shape=[
            jax.ShapeDtypeStruct((NSTEP, 1, T), jnp.int32),
            jax.ShapeDtypeStruct((B, D, HW), jnp.float32),
            jax.ShapeDtypeStruct((1, 1), jnp.float32),
            jax.ShapeDtypeStruct((1, 1), jnp.float32),
        ],
        scratch_shapes=[
            pltpu.VMEM((K,), jnp.float32),
            pltpu.SMEM((1,), jnp.float32),
        ],
    )(xr, embedding)
    enc0 = _sc_fill()
    enc_ref = jax.new_ref(enc0)
    _sc_scatter(enc_ref, idx3.reshape(NTOK))
    enc = enc_ref[...].reshape(NTOK, K)
    quantized = q.reshape(B, D, 32, 32)
    return (loss[0, 0], quantized, perp[0, 0], enc)


# speedup vs baseline: 1.0021x; 1.0021x over previous
import functools

import jax
import jax.numpy as jnp
from jax import lax
from jax.experimental import pallas as pl
from jax.experimental.pallas import tpu as pltpu
from jax.experimental.pallas import tpu_sc as plsc

K = 1024
D = 64
B = 16
HW = 1024
T = 1024
C = HW // T
NTOK = B * HW
NSTEP = B * C
COMMIT = 0.25

NC = 2
NS = 16
NW = NC * NS
TPT = NTOK // NW
M = NTOK * K
EPT = M // NW
ZCHUNK = 16384
NZDMA = EPT // ZCHUNK


def _vq_tc_body(x_ref, emb_ref, idx_ref, q_ref, loss_ref, perp_ref,
                counts_ref, sse_ref):
    i = pl.program_id(0)

    @pl.when(i == 0)
    def _init():
        counts_ref[...] = jnp.zeros_like(counts_ref)
        sse_ref[0] = 0.0

    x = x_ref[0]
    x2 = jnp.sum(x ** 2, axis=0)
    xd = x + x
    emb = emb_ref[...]
    e2 = jnp.sum(emb ** 2, axis=1)
    xe2 = lax.dot_general(xd, emb, (((0,), (1,)), ((), ())),
                          preferred_element_type=jnp.float32)
    scores = (x2[:, None] + e2[None, :]) - xe2
    minval = jnp.min(scores, axis=1)
    iota_k = lax.broadcasted_iota(jnp.int32, (T, K), 1)
    idx = jnp.min(jnp.where(scores == minval[:, None], iota_k, K), axis=1)
    idx_ref[0, 0] = idx
    enc = (iota_k == idx[:, None]).astype(jnp.float32)
    counts_ref[...] += jnp.sum(enc, axis=0)
    qT = lax.dot_general(emb, enc, (((0,), (1,)), ((), ())),
                         preferred_element_type=jnp.float32)
    q_ref[0] = qT
    diff = qT - x
    sse_ref[0] += jnp.sum(diff * diff)

    @pl.when(i == NSTEP - 1)
    def _fini():
        loss_ref[0, 0] = (1.0 + COMMIT) * sse_ref[0] / (NTOK * D)
        avg = counts_ref[...] * (1.0 / NTOK)
        perp_ref[0, 0] = jnp.exp(-jnp.sum(avg * jnp.log(avg + 1e-10)))


_SC_MESH = plsc.VectorSubcoreMesh(core_axis_name="c", subcore_axis_name="s")


@functools.partial(
    pl.kernel,
    out_type=jax.ShapeDtypeStruct((M,), jnp.float32),
    mesh=_SC_MESH,
    scratch_types=[pltpu.VMEM((ZCHUNK,), jnp.float32)],
)
def _sc_fill(out_hbm, zbuf):
    wid = lax.axis_index("s") * NC + lax.axis_index("c")
    base = wid * EPT

    def _zero(j, carry):
        zbuf[pl.ds(j * 16, 16)] = jnp.zeros((16,), jnp.float32)
        return carry

    lax.fori_loop(0, ZCHUNK // 16, _zero, 0)

    def _dma(t, carry):
        pltpu.sync_copy(zbuf, out_hbm.at[pl.ds(base + t * ZCHUNK, ZCHUNK)])
        return carry

    lax.fori_loop(0, NZDMA, _dma, 0)


@functools.partial(
    pl.kernel,
    out_type=(),
    mesh=_SC_MESH,
    scratch_types=[
        pltpu.VMEM((TPT,), jnp.int32),
        pltpu.VMEM((4, 128), jnp.int32),
        pltpu.VMEM((128,), jnp.float32),
        pltpu.SemaphoreType.DMA,
    ],
)
def _sc_scatter(enc_ref, idx_hbm, idxbuf, offbuf, ones, sem):
    wid = lax.axis_index("s") * NC + lax.axis_index("c")
    base = wid * TPT
    pltpu.sync_copy(idx_hbm.at[pl.ds(base, TPT)], idxbuf)
    for j in range(8):
        ones[pl.ds(j * 16, 16)] = jnp.ones((16,), jnp.float32)
    lane_k = lax.iota(jnp.int32, 16) * K
    for j in range(TPT // 16):
        iv = idxbuf[pl.ds(j * 16, 16)]
        off = iv + lane_k + ((base + j * 16) * K)
        offbuf[j // 8, pl.ds((j % 8) * 16, 16)] = off
    copies = [pltpu.async_copy(ones, enc_ref.at[offbuf.at[c]], sem)
              for c in range(4)]
    for cp in copies:
        cp.wait()


def kernel(inputs, embedding):
    xr = inputs.reshape(B, D, HW)
    idx3, q, loss, perp = pl.pallas_call(
        _vq_tc_body,
        grid=(NSTEP,),
        in_specs=[
            pl.BlockSpec((1, D, T), lambda i: (i // C, 0, i % C)),
            pl.BlockSpec((K, D), lambda i: (0, 0)),
        ],
        out_specs=[
            pl.BlockSpec((1, 1, T), lambda i: (i, 0, 0)),
            pl.BlockSpec((1, D, T), lambda i: (i // C, 0, i % C)),
            pl.BlockSpec((1, 1), lambda i: (0, 0), memory_space=pltpu.SMEM),
            pl.BlockSpec((1, 1), lambda i: (0, 0), memory_space=pltpu.SMEM),
        ],
        out_shape=[
            jax.ShapeDtypeStruct((NSTEP, 1, T), jnp.int32),
            jax.ShapeDtypeStruct((B, D, HW), jnp.float32),
            jax.ShapeDtypeStruct((1, 1), jnp.float32),
            jax.ShapeDtypeStruct((1, 1), jnp.float32),
        ],
        scratch_shapes=[
            pltpu.VMEM((K,), jnp.float32),
            pltpu.SMEM((1,), jnp.float32),
        ],
    )(xr, embedding)
    enc0 = _sc_fill()
    enc_ref = jax.new_ref(enc0)
    _sc_scatter(enc_ref, idx3.reshape(NTOK))
    enc = jax.freeze(enc_ref).reshape(NTOK, K)
    quantized = q.reshape(B, D, 32, 32)
    return (loss[0, 0], quantized, perp[0, 0], enc)


# speedup vs baseline: 2.7047x; 2.6989x over previous
import jax
import jax.numpy as jnp
from jax import lax
from jax.experimental import pallas as pl
from jax.experimental.pallas import tpu as pltpu

K = 1024
D = 64
B = 16
HW = 1024
T = 1024
C = HW // T
NTOK = B * HW
NSTEP = B * C
COMMIT = 0.25


def _vq_body(x_ref, emb_ref, enc_ref, q_ref, loss_ref, perp_ref,
             counts_ref, sse_ref):
    i = pl.program_id(0)

    @pl.when(i == 0)
    def _init():
        counts_ref[...] = jnp.zeros_like(counts_ref)
        sse_ref[0] = 0.0

    x = x_ref[0]
    x2 = jnp.sum(x ** 2, axis=0)
    xd = x + x
    emb = emb_ref[...]
    e2 = jnp.sum(emb ** 2, axis=1)
    xe2 = lax.dot_general(xd, emb, (((0,), (1,)), ((), ())),
                          preferred_element_type=jnp.float32)
    scores = (x2[:, None] + e2[None, :]) - xe2
    idx = jnp.argmin(scores, axis=1).astype(jnp.int32)
    iota_k = lax.broadcasted_iota(jnp.int32, (T, K), 1)
    enc = (iota_k == idx[:, None]).astype(jnp.float32)
    enc_ref[...] = enc
    counts_ref[...] += jnp.sum(enc, axis=0)
    qT = lax.dot_general(emb, enc, (((0,), (1,)), ((), ())),
                         preferred_element_type=jnp.float32)
    q_ref[0] = qT
    diff = qT - x
    sse_ref[0] += jnp.sum(diff * diff)

    @pl.when(i == NSTEP - 1)
    def _fini():
        loss_ref[0, 0] = (1.0 + COMMIT) * sse_ref[0] / (NTOK * D)
        avg = counts_ref[...] * (1.0 / NTOK)
        perp_ref[0, 0] = jnp.exp(-jnp.sum(avg * jnp.log(avg + 1e-10)))


def kernel(inputs, embedding):
    xr = inputs.reshape(B, D, HW)
    enc, q, loss, perp = pl.pallas_call(
        _vq_body,
        grid=(NSTEP,),
        in_specs=[
            pl.BlockSpec((1, D, T), lambda i: (i // C, 0, i % C)),
            pl.BlockSpec((K, D), lambda i: (0, 0)),
        ],
        out_specs=[
            pl.BlockSpec((T, K), lambda i: (i, 0)),
            pl.BlockSpec((1, D, T), lambda i: (i // C, 0, i % C)),
            pl.BlockSpec((1, 1), lambda i: (0, 0), memory_space=pltpu.SMEM),
            pl.BlockSpec((1, 1), lambda i: (0, 0), memory_space=pltpu.SMEM),
        ],
        out_shape=[
            jax.ShapeDtypeStruct((NTOK, K), jnp.float32),
            jax.ShapeDtypeStruct((B, D, HW), jnp.float32),
            jax.ShapeDtypeStruct((1, 1), jnp.float32),
            jax.ShapeDtypeStruct((1, 1), jnp.float32),
        ],
        scratch_shapes=[
            pltpu.VMEM((K,), jnp.float32),
            pltpu.SMEM((1,), jnp.float32),
        ],
    )(xr, embedding)
    quantized = q.reshape(B, D, 32, 32)
    return (loss[0, 0], quantized, perp[0, 0], enc)
